# baseline (speedup 1.0000x reference)
import functools

import jax
import jax.numpy as jnp
from jax import lax
from jax.experimental import pallas as pl
from jax.experimental.pallas import tpu as pltpu
from jax.experimental.pallas import tpu_sc as plsc

N = 8192
H = 128
F_IN = 128
HEADS = 3
E = 262144

_NTILES = 32
_CHUNK = 128
_RPS = N // 16

_SC_MESH = plsc.VectorSubcoreMesh(core_axis_name="c", subcore_axis_name="s")


_HC = H // 2
_TPC = 16
_NCHT = 144
_NE = _TPC * _CHUNK * _NCHT


def _wsegsum_factory():
    ncht = _NCHT

    dnums = lax.GatherDimensionNumbers(
        offset_dims=(), collapsed_slice_dims=(0,), start_index_map=(0,))

    def _bcast(w16, i):
        idx = jnp.full((16, 1), i, jnp.int32)
        return lax.gather(w16, idx, dnums, (1,),
                          mode=lax.GatherScatterMode.PROMISE_IN_BOUNDS)

    def _mul_rows(rows, wts_v, j):
        for g in range(_CHUNK // 16):
            w16 = wts_v[j, pl.ds(g * 16, 16)]
            for i in range(16):
                wv = _bcast(w16, i)
                r = g * 16 + i
                for cb in range(_HC // 16):
                    sl = pl.ds(cb * 16, 16)
                    rows[r, sl] = rows[r, sl] * wv

    def body(h_hbm, src_hbm, dst_hbm, wts_hbm, out_hbm,
             src_v, dst_v, wts_v, rows_a, rows_b, zeros_v, acc,
             sem_a, sem_b, sem_sa, sem_sb):
        c = lax.axis_index("c")
        s = lax.axis_index("s")
        cbase = s * ncht

        pltpu.sync_copy(src_hbm.at[pl.ds(cbase, ncht)], src_v)
        pltpu.sync_copy(dst_hbm.at[pl.ds(cbase, ncht)], dst_v)
        pltpu.sync_copy(wts_hbm.at[pl.ds(cbase, ncht)], wts_v)

        off = c * N

        def orow(i, carry):
            for g in range(_CHUNK // 16):
                sl = pl.ds(g * 16, 16)
                src_v[i, sl] = src_v[i, sl] + off
            return carry
        lax.fori_loop(0, ncht, orow, 0)

        def zrow(i, carry):
            for j in range(_HC // 16):
                zeros_v[i, pl.ds(j * 16, 16)] = jnp.zeros((16,), jnp.float32)
            return carry
        lax.fori_loop(0, _CHUNK, zrow, 0)
        for k in range(_RPS // _CHUNK):
            pltpu.sync_copy(zeros_v,
                            acc.at[pl.ds(s * _RPS + k * _CHUNK, _CHUNK)])
        plsc.subcore_barrier()

        pltpu.async_copy(h_hbm.at[src_v.at[0]], rows_a, sem_a)
        pltpu.async_copy(zeros_v, acc.at[dst_v.at[0]], sem_sb, add=True)

        def step(k, carry):
            j0 = 2 * k
            j1 = 2 * k + 1
            pltpu.make_async_copy(h_hbm.at[src_v.at[0]], rows_a, sem_a).wait()
            _mul_rows(rows_a, wts_v, j0)
            pltpu.make_async_copy(zeros_v, acc.at[dst_v.at[0]], sem_sb).wait()
            pltpu.async_copy(h_hbm.at[src_v.at[j1]], rows_b, sem_b)
            pltpu.async_copy(rows_a, acc.at[dst_v.at[j0]], sem_sa, add=True)
            pltpu.make_async_copy(h_hbm.at[src_v.at[0]], rows_b, sem_b).wait()
            _mul_rows(rows_b, wts_v, j1)
            pltpu.make_async_copy(zeros_v, acc.at[dst_v.at[0]], sem_sa).wait()
            jn = jnp.minimum(j0 + 2, ncht - 1)
            pltpu.async_copy(h_hbm.at[src_v.at[jn]], rows_a, sem_a)
            pltpu.async_copy(rows_b, acc.at[dst_v.at[j1]], sem_sb, add=True)
            return carry

        lax.fori_loop(0, ncht // 2, step, 0)
        pltpu.make_async_copy(h_hbm.at[src_v.at[0]], rows_a, sem_a).wait()
        pltpu.make_async_copy(zeros_v, acc.at[dst_v.at[0]], sem_sb).wait()

        plsc.subcore_barrier()
        for k in range(_RPS // _CHUNK):
            r0 = s * _RPS + k * _CHUNK
            pltpu.sync_copy(acc.at[pl.ds(r0, _CHUNK)],
                            out_hbm.at[c, pl.ds(r0, _CHUNK)])

    return pl.kernel(
        body,
        out_type=jax.ShapeDtypeStruct((2, N, _HC), jnp.float32),
        mesh=_SC_MESH,
        scratch_types=[
            pltpu.VMEM((ncht, _CHUNK), jnp.int32),
            pltpu.VMEM((ncht, _CHUNK), jnp.int32),
            pltpu.VMEM((ncht, _CHUNK), jnp.float32),
            pltpu.VMEM((_CHUNK, _HC), jnp.float32),
            pltpu.VMEM((_CHUNK, _HC), jnp.float32),
            pltpu.VMEM((_CHUNK, _HC), jnp.float32),
            pltpu.VMEM_SHARED((N, _HC), jnp.float32),
            pltpu.SemaphoreType.DMA,
            pltpu.SemaphoreType.DMA,
            pltpu.SemaphoreType.DMA,
            pltpu.SemaphoreType.DMA,
        ],
        compiler_params=pltpu.CompilerParams(use_tc_tiling_on_sc=False),
    )


_WSEGSUM = _wsegsum_factory()


def _wsegsum(h, src2, dst2, wts2):
    h2 = jnp.concatenate([h[:, :_HC], h[:, _HC:]], axis=0)
    p = _WSEGSUM(h2, src2, dst2, wts2)
    return jnp.concatenate([p[0], p[1]], axis=1)


def _mm_relu_kernel(x_ref, w_ref, b_ref, o_ref):
    o_ref[...] = jnp.maximum(
        jnp.dot(x_ref[...], w_ref[...], preferred_element_type=jnp.float32)
        + b_ref[...],
        0.0,
    )


def _emb(x, W, b):
    blk = N // 8
    return pl.pallas_call(
        _mm_relu_kernel,
        grid=(8,),
        in_specs=[
            pl.BlockSpec((blk, F_IN), lambda i: (i, 0)),
            pl.BlockSpec((F_IN, H), lambda i: (0, 0)),
            pl.BlockSpec((1, H), lambda i: (0, 0)),
        ],
        out_specs=pl.BlockSpec((blk, H), lambda i: (i, 0)),
        out_shape=jax.ShapeDtypeStruct((N, H), jnp.float32),
    )(x, W, b.reshape(1, H))


def _bn(x, gamma, beta):
    mu = jnp.mean(x, axis=0)
    var = jnp.var(x, axis=0)
    return (x - mu) / jnp.sqrt(var + 1e-5) * gamma + beta


def _cheb(x, src2, dst2, norm2, W, b):
    def prop(h):
        return _wsegsum(h, src2, dst2, norm2)

    Tx0 = x
    Tx1 = prop(Tx0)
    Tx2 = 2.0 * prop(Tx1) - Tx0
    return Tx0 @ W[0] + Tx1 @ W[1] + Tx2 @ W[2] + b


def _gat(x, s2, d2, W, att_src, att_dst, bias):
    s = s2.reshape(-1)[:E + N]
    d = d2.reshape(-1)[:E + N]
    h = (x @ W).reshape(N, HEADS, H)
    a_s = jnp.sum(h * att_src[None, :, :], axis=-1)
    a_d = jnp.sum(h * att_dst[None, :, :], axis=-1)
    e = a_s[s] + a_d[d]
    e = jnp.where(e > 0, e, 0.2 * e)
    m = jax.ops.segment_max(e, d, num_segments=N)
    m = jnp.where(jnp.isfinite(m), m, 0.0)
    ex = jnp.exp(e - m[d])
    ssum = jax.ops.segment_sum(ex, d, num_segments=N)
    alpha = ex / (ssum[d] + 1e-16)
    wpad = jnp.zeros((_NE - (E + N),), jnp.float32)
    outs = []
    for k in range(HEADS):
        hk = h[:, k, :]
        wk = jnp.concatenate([alpha[:, k], wpad]).reshape(s2.shape)
        outs.append(_wsegsum(hk, s2, d2, wk))
    return jnp.concatenate(outs, axis=1) + bias


def _seq(x, s2, d2, p, i):
    g = _gat(x, s2, d2, p[f'gat{i}_W'], p[f'gat{i}_att_src'],
             p[f'gat{i}_att_dst'], p[f'gat{i}_bias'])
    g = _bn(g, p[f'bn{i}_gamma'], p[f'bn{i}_beta'])
    return g @ p[f'lin{i}_W'] + p[f'lin{i}_b']


def kernel(x, params, edge_index):
    src = edge_index[0]
    dst = edge_index[1]
    loop = jnp.arange(N, dtype=src.dtype)
    zc = jnp.zeros((_NE - E,), src.dtype)
    zg = jnp.zeros((_NE - (E + N),), src.dtype)
    src2 = jnp.concatenate([src, zc]).reshape(_NE // _CHUNK, _CHUNK)
    dst2 = jnp.concatenate([dst, zc]).reshape(_NE // _CHUNK, _CHUNK)
    s2 = jnp.concatenate([src, loop, zg]).reshape(_NE // _CHUNK, _CHUNK)
    d2 = jnp.concatenate([dst, loop, zg]).reshape(_NE // _CHUNK, _CHUNK)
    w_e = jnp.ones(src.shape[0], dtype=jnp.float32)
    deg = jax.ops.segment_sum(w_e, src, num_segments=N)
    dinv = jnp.where(deg > 0, 1.0 / jnp.sqrt(deg), 0.0)
    norm2 = jnp.concatenate(
        [-dinv[src] * dinv[dst], jnp.zeros((_NE - E,), jnp.float32)]
    ).reshape(src2.shape)
    p = params
    y = _emb(x, p['W_emb'], p['b_emb'])
    y = _cheb(y, src2, dst2, norm2, p['cheb_W0'], p['cheb_b0'])
    y = _bn(y, p['bn_gamma'], p['bn_beta'])
    y_gat = _seq(y, s2, d2, p, 0)
    y = jnp.maximum(y, 0.0)
    y1 = _cheb(y, src2, dst2, norm2, p['cheb_W1'], p['cheb_b1'])
    y1 = _bn(y1, p['bn_gamma'], p['bn_beta'])
    y1_gat = _seq(y1, s2, d2, p, 1)
    y1 = jnp.maximum(y1, 0.0)
    y2 = _cheb(y1, src2, dst2, norm2, p['cheb_W2'], p['cheb_b2'])
    y2 = _bn(y2, p['bn_gamma'], p['bn_beta'])
    y2_gat = _seq(y2, s2, d2, p, 2)
    return y2 + y_gat + y1_gat + y2_gat


# speedup vs baseline: 1.9719x; 1.9719x over previous
import functools

import jax
import jax.numpy as jnp
from jax import lax
from jax.experimental import pallas as pl
from jax.experimental.pallas import tpu as pltpu
from jax.experimental.pallas import tpu_sc as plsc

N = 8192
H = 128
F_IN = 128
HEADS = 3
E = 262144

_NTILES = 32
_CHUNK = 128
_RPS = N // 16

_SC_MESH = plsc.VectorSubcoreMesh(core_axis_name="c", subcore_axis_name="s")


_HC = H // 2
_TPC = 16
_NCHT = 144
_NE = _TPC * _CHUNK * _NCHT


def _wsegsum_factory():
    ncht = _NCHT

    dnums = lax.GatherDimensionNumbers(
        offset_dims=(), collapsed_slice_dims=(0,), start_index_map=(0,))

    def _bcast(w16, i):
        idx = jnp.full((16, 1), i, jnp.int32)
        return lax.gather(w16, idx, dnums, (1,),
                          mode=lax.GatherScatterMode.PROMISE_IN_BOUNDS)

    def _mul_rows(rows, src_v, dst_v, wts_v, ta_v, tb_v, j):
        for g in range(_CHUNK // 16):
            sl16 = pl.ds(g * 16, 16)
            w16 = wts_v[j, sl16]
            ga = plsc.load_gather(ta_v, [dst_v[j, sl16]])
            gb = plsc.load_gather(tb_v, [src_v[j, sl16]])
            w16 = w16 * ga * gb
            for i in range(16):
                wv = _bcast(w16, i)
                r = g * 16 + i
                for cb in range(_HC // 16):
                    sl = pl.ds(cb * 16, 16)
                    rows[r, sl] = rows[r, sl] * wv

    def body(h_hbm, src_hbm, dst_hbm, wts_hbm, ta_hbm, tb_hbm, out_hbm,
             src_v, dst_v, wts_v, ta_v, tb_v, rows_a, acc, sem_a):
        c = lax.axis_index("c")
        s = lax.axis_index("s")
        cbase = s * ncht
        off = c * N

        pltpu.sync_copy(ta_hbm, ta_v)
        pltpu.sync_copy(tb_hbm, tb_v)

        def zrow(i, carry):
            for j in range(_HC // 16):
                rows_a[i, pl.ds(j * 16, 16)] = jnp.zeros((16,), jnp.float32)
            return carry
        lax.fori_loop(0, _CHUNK, zrow, 0)
        for k in range(_RPS // _CHUNK):
            pltpu.sync_copy(rows_a,
                            acc.at[pl.ds(s * _RPS + k * _CHUNK, _CHUNK)])
        plsc.subcore_barrier()

        def outer(ko, carry):
            base = cbase + ko * 16
            pltpu.sync_copy(src_hbm.at[pl.ds(base, 16)], src_v)
            pltpu.sync_copy(dst_hbm.at[pl.ds(base, 16)], dst_v)
            pltpu.sync_copy(wts_hbm.at[pl.ds(base, 16)], wts_v)

            def offr(i, cr):
                for g in range(_CHUNK // 16):
                    sl = pl.ds(g * 16, 16)
                    src_v[i, sl] = src_v[i, sl] + off
                return cr
            lax.fori_loop(0, 16, offr, 0)

            def chunk(jj, cr):
                pltpu.async_copy(h_hbm.at[src_v.at[jj]], rows_a, sem_a).wait()
                _mul_rows(rows_a, src_v, dst_v, wts_v, ta_v, tb_v, jj)
                pltpu.sync_copy(rows_a, acc.at[dst_v.at[jj]], add=True)
                return cr
            lax.fori_loop(0, 16, chunk, 0)
            return carry

        lax.fori_loop(0, ncht // 16, outer, 0)

        plsc.subcore_barrier()
        for k in range(_RPS // _CHUNK):
            r0 = s * _RPS + k * _CHUNK
            pltpu.sync_copy(acc.at[pl.ds(r0, _CHUNK)],
                            out_hbm.at[c, pl.ds(r0, _CHUNK)])

    return pl.kernel(
        body,
        out_type=jax.ShapeDtypeStruct((2, N, _HC), jnp.float32),
        mesh=_SC_MESH,
        scratch_types=[
            pltpu.VMEM((16, _CHUNK), jnp.int32),
            pltpu.VMEM((16, _CHUNK), jnp.int32),
            pltpu.VMEM((16, _CHUNK), jnp.float32),
            pltpu.VMEM((N,), jnp.float32),
            pltpu.VMEM((2 * N,), jnp.float32),
            pltpu.VMEM((_CHUNK, _HC), jnp.float32),
            pltpu.VMEM_SHARED((N, _HC), jnp.float32),
            pltpu.SemaphoreType.DMA,
        ],
        compiler_params=pltpu.CompilerParams(
            use_tc_tiling_on_sc=False, needs_layout_passes=False),
    )


_WSEGSUM = _wsegsum_factory()


def _wsegsum(h, src2, dst2, wts2, ta, tb2):
    h2 = jnp.concatenate([h[:, :_HC], h[:, _HC:]], axis=0)
    p = _WSEGSUM(h2, src2, dst2, wts2, ta, tb2)
    return jnp.concatenate([p[0], p[1]], axis=1)


def _edge_softmax_factory():
    ncht = _NE // (_NTILES * _CHUNK)
    n3 = 3 * N

    def body(s_hbm, d_hbm, asad_hbm, ex_hbm, ss_hbm,
             src_v, dst_v, tb, hist, exb, sem):
        c = lax.axis_index("c")
        s = lax.axis_index("s")
        tid = s * 2 + c
        cbase = tid * ncht

        pltpu.sync_copy(s_hbm.at[pl.ds(cbase, ncht)], src_v)
        pltpu.sync_copy(d_hbm.at[pl.ds(cbase, ncht)], dst_v)
        pltpu.sync_copy(asad_hbm, tb)

        def zh(i, carry):
            z = jnp.zeros((16,), jnp.float32)
            for k in range(3):
                hist[k, pl.ds(i * 16, 16)] = z
            return carry
        lax.fori_loop(0, N // 16, zh, 0)

        iota = lax.iota(jnp.int32, 16)
        nreal = jnp.int32(E + N)

        def outer(ko, carry):
            for kc in range(8):
                j = ko * 8 + kc
                for g in range(8):
                    sl16 = pl.ds(g * 16, 16)
                    s16 = src_v[j, sl16]
                    d16 = dst_v[j, sl16]
                    gbase = (cbase + j) * _CHUNK + g * 16
                    msk = (gbase + iota) < nreal
                    for k in range(3):
                        av = plsc.load_gather(tb.at[k], [s16])
                        bv = plsc.load_gather(tb.at[3 + k], [d16])
                        ev = av + bv
                        ev = jnp.where(ev > 0, ev, 0.2 * ev)
                        xv = jnp.where(msk, jnp.exp(ev), 0.0)
                        plsc.addupdate_scatter(hist.at[k], [d16], xv)
                        exb[k, kc, sl16] = xv
            for k in range(3):
                pltpu.sync_copy(
                    exb.at[k],
                    ex_hbm.at[k, pl.ds(cbase + ko * 8, 8)])
            return carry

        lax.fori_loop(0, ncht // 8, outer, 0)

        pltpu.sync_copy(hist, ss_hbm.at[tid])

    return pl.kernel(
        body,
        out_type=(
            jax.ShapeDtypeStruct((3, _NE // _CHUNK, _CHUNK), jnp.float32),
            jax.ShapeDtypeStruct((_NTILES, 3, N), jnp.float32),
        ),
        mesh=_SC_MESH,
        scratch_types=[
            pltpu.VMEM((ncht, _CHUNK), jnp.int32),
            pltpu.VMEM((ncht, _CHUNK), jnp.int32),
            pltpu.VMEM((6, N), jnp.float32),
            pltpu.VMEM((3, N), jnp.float32),
            pltpu.VMEM((3, 8, _CHUNK), jnp.float32),
            pltpu.SemaphoreType.DMA,
        ],
        compiler_params=pltpu.CompilerParams(use_tc_tiling_on_sc=False, needs_layout_passes=False),
    )


def _deg_factory():
    ncht = _NE // (_NTILES * _CHUNK)

    def body(s_hbm, deg_hbm, src_v, hist, sem):
        c = lax.axis_index("c")
        s = lax.axis_index("s")
        tid = s * 2 + c
        cbase = tid * ncht

        pltpu.sync_copy(s_hbm.at[pl.ds(cbase, ncht)], src_v)

        def zh(i, carry):
            hist[pl.ds(i * 16, 16)] = jnp.zeros((16,), jnp.float32)
            return carry
        lax.fori_loop(0, N // 16, zh, 0)

        iota = lax.iota(jnp.int32, 16)
        nreal = jnp.int32(E)

        def outer(j, carry):
            for g in range(8):
                sl16 = pl.ds(g * 16, 16)
                s16 = src_v[j, sl16]
                gbase = (cbase + j) * _CHUNK + g * 16
                ones = jnp.where((gbase + iota) < nreal, 1.0, 0.0)
                plsc.addupdate_scatter(hist, [s16], ones)
            return carry

        lax.fori_loop(0, ncht, outer, 0)
        pltpu.sync_copy(hist, deg_hbm.at[tid])

    return pl.kernel(
        body,
        out_type=jax.ShapeDtypeStruct((_NTILES, N), jnp.float32),
        mesh=_SC_MESH,
        scratch_types=[
            pltpu.VMEM((ncht, _CHUNK), jnp.int32),
            pltpu.VMEM((N,), jnp.float32),
            pltpu.SemaphoreType.DMA,
        ],
        compiler_params=pltpu.CompilerParams(use_tc_tiling_on_sc=False, needs_layout_passes=False),
    )


_EDGE_SOFTMAX = _edge_softmax_factory()
_DEG = _deg_factory()


def _mm_relu_kernel(x_ref, w_ref, b_ref, o_ref):
    o_ref[...] = jnp.maximum(
        jnp.dot(x_ref[...], w_ref[...], preferred_element_type=jnp.float32)
        + b_ref[...],
        0.0,
    )


def _emb(x, W, b):
    blk = N // 8
    return pl.pallas_call(
        _mm_relu_kernel,
        grid=(8,),
        in_specs=[
            pl.BlockSpec((blk, F_IN), lambda i: (i, 0)),
            pl.BlockSpec((F_IN, H), lambda i: (0, 0)),
            pl.BlockSpec((1, H), lambda i: (0, 0)),
        ],
        out_specs=pl.BlockSpec((blk, H), lambda i: (i, 0)),
        out_shape=jax.ShapeDtypeStruct((N, H), jnp.float32),
    )(x, W, b.reshape(1, H))


def _bn(x, gamma, beta):
    mu = jnp.mean(x, axis=0)
    var = jnp.var(x, axis=0)
    return (x - mu) / jnp.sqrt(var + 1e-5) * gamma + beta


def _cheb(x, src2, dst2, wts_cheb, dinv, dinv2, W, b):
    def prop(h):
        return _wsegsum(h, src2, dst2, wts_cheb, dinv, dinv2)

    Tx0 = x
    Tx1 = prop(Tx0)
    Tx2 = 2.0 * prop(Tx1) - Tx0
    return Tx0 @ W[0] + Tx1 @ W[1] + Tx2 @ W[2] + b


def _gat(x, s2, d2, ones2, W, att_src, att_dst, bias):
    h = x @ W
    hr = h.reshape(N, HEADS, H)
    a_s = jnp.sum(hr * att_src[None, :, :], axis=-1)
    a_d = jnp.sum(hr * att_dst[None, :, :], axis=-1)
    asad = jnp.concatenate([a_s.T, a_d.T], axis=0)
    ex, ssp = _EDGE_SOFTMAX(s2, d2, asad)
    rinv = 1.0 / ssp.sum(axis=0)
    outs = []
    for k in range(HEADS):
        outs.append(_wsegsum(hr[:, k, :], s2, d2, ex[k], rinv[k], ones2))
    return jnp.concatenate(outs, axis=1) + bias


def _seq(x, s2, d2, ones2, p, i):
    g = _gat(x, s2, d2, ones2, p[f'gat{i}_W'], p[f'gat{i}_att_src'],
             p[f'gat{i}_att_dst'], p[f'gat{i}_bias'])
    g = _bn(g, p[f'bn{i}_gamma'], p[f'bn{i}_beta'])
    return g @ p[f'lin{i}_W'] + p[f'lin{i}_b']


def kernel(x, params, edge_index):
    src = edge_index[0]
    dst = edge_index[1]
    loop = jnp.arange(N, dtype=src.dtype)
    zc = jnp.zeros((_NE - E,), src.dtype)
    zg = jnp.zeros((_NE - (E + N),), src.dtype)
    src2 = jnp.concatenate([src, zc]).reshape(_NE // _CHUNK, _CHUNK)
    dst2 = jnp.concatenate([dst, zc]).reshape(_NE // _CHUNK, _CHUNK)
    s2 = jnp.concatenate([src, loop, zg]).reshape(_NE // _CHUNK, _CHUNK)
    d2 = jnp.concatenate([dst, loop, zg]).reshape(_NE // _CHUNK, _CHUNK)
    degp = _DEG(src2)
    deg = degp.sum(axis=0)
    dinv = jnp.where(deg > 0, 1.0 / jnp.sqrt(deg), 0.0)
    dinv2 = jnp.concatenate([dinv, dinv])
    ones2 = jnp.ones((2 * N,), jnp.float32)
    wts_cheb = jnp.concatenate(
        [jnp.full((E,), -1.0, jnp.float32), jnp.zeros((_NE - E,), jnp.float32)]
    ).reshape(src2.shape)
    p = params
    y = _emb(x, p['W_emb'], p['b_emb'])
    y = _cheb(y, src2, dst2, wts_cheb, dinv, dinv2, p['cheb_W0'], p['cheb_b0'])
    y = _bn(y, p['bn_gamma'], p['bn_beta'])
    y_gat = _seq(y, s2, d2, ones2, p, 0)
    y = jnp.maximum(y, 0.0)
    y1 = _cheb(y, src2, dst2, wts_cheb, dinv, dinv2, p['cheb_W1'], p['cheb_b1'])
    y1 = _bn(y1, p['bn_gamma'], p['bn_beta'])
    y1_gat = _seq(y1, s2, d2, ones2, p, 1)
    y1 = jnp.maximum(y1, 0.0)
    y2 = _cheb(y1, src2, dst2, wts_cheb, dinv, dinv2, p['cheb_W2'], p['cheb_b2'])
    y2 = _bn(y2, p['bn_gamma'], p['bn_beta'])
    y2_gat = _seq(y2, s2, d2, ones2, p, 2)
    return y2 + y_gat + y1_gat + y2_gat
